# baseline (speedup 1.0000x reference)
import functools

import jax
import jax.numpy as jnp
from jax import lax
from jax.experimental import pallas as pl
from jax.experimental.pallas import tpu as pltpu
from jax.experimental.pallas import tpu_sc as plsc

_B = 2
_S = 4096
_T = _B * _S
_D = 1024
_F = 4096
_NT = 4

_TILE = 256
_P = _T + _NT * _TILE
_NUM_TILES = _P // _TILE
_FB = 2048
_NF = _F // _FB

_NC = 2
_NS = 16
_NW = _NC * _NS
_CHUNK = 32


def _sc_gather(table, idx, chunk):
    v, d = table.shape
    n = idx.shape[0]
    rows_per = n // _NW
    n_chunks = rows_per // chunk
    mesh = plsc.VectorSubcoreMesh(core_axis_name="c", subcore_axis_name="s")

    @functools.partial(
        pl.kernel,
        mesh=mesh,
        out_type=jax.ShapeDtypeStruct((n, d), table.dtype),
        scratch_types=[
            pltpu.VMEM((2, chunk), jnp.int32),
            pltpu.VMEM((2, chunk, d), table.dtype),
            pltpu.SemaphoreType.DMA,
            pltpu.SemaphoreType.DMA,
        ],
    )
    def gather_k(table_hbm, idx_hbm, out_hbm, idx_v, rows_v, sem0, sem1):
        wid = lax.axis_index("s") * _NC + lax.axis_index("c")
        base = wid * rows_per
        sems = (sem0, sem1)
        copies = [None, None]

        def start(c):
            b = c % 2
            pltpu.sync_copy(idx_hbm.at[pl.ds(base + c * chunk, chunk)],
                            idx_v.at[b])
            copies[b] = pltpu.async_copy(table_hbm.at[idx_v.at[b]],
                                         rows_v.at[b], sems[b])

        start(0)
        for c in range(n_chunks):
            b = c % 2
            if c + 1 < n_chunks:
                start(c + 1)
            copies[b].wait()
            pltpu.sync_copy(rows_v.at[b],
                            out_hbm.at[pl.ds(base + c * chunk, chunk)])

    return gather_k(table, idx)


def _cast_body(x_ref, o_ref):
    xb = x_ref[...].astype(jnp.bfloat16)
    u = pltpu.bitcast(xb, jnp.uint16)
    lo = u[:, :_D // 2].astype(jnp.uint32)
    hi = u[:, _D // 2:].astype(jnp.uint32)
    o_ref[...] = (lo | (hi << 16)).astype(jnp.int32)


def _detile_cast(x2):
    blk = 1024
    return pl.pallas_call(
        _cast_body,
        grid=(_T // blk,),
        in_specs=[pl.BlockSpec((blk, _D), lambda i: (i, 0))],
        out_specs=pl.BlockSpec((blk, _D // 2), lambda i: (i, 0)),
        out_shape=jax.ShapeDtypeStruct((_T, _D // 2), jnp.int32),
    )(x2)


def _ffn_body(tt_ref, valid_ref, xs_ref, w1_ref, b1_ref, w2_ref, b2_ref,
              out_ref):
    m = pl.program_id(0)
    f = pl.program_id(1)

    @pl.when(valid_ref[m] == 1)
    def _():
        wu = pltpu.bitcast(xs_ref[...], jnp.uint32)
        lo = pltpu.bitcast((wu & 0xffff).astype(jnp.uint16), jnp.bfloat16)
        hi = pltpu.bitcast((wu >> 16).astype(jnp.uint16), jnp.bfloat16)
        xb = jnp.concatenate([lo, hi], axis=1)
        a = jnp.dot(xb, w1_ref[0],
                    preferred_element_type=jnp.float32) + b1_ref[0]
        h = 0.5 * a * (1.0 + lax.erf(a * 0.7071067811865476))
        contrib = jnp.dot(h.astype(jnp.bfloat16), w2_ref[0],
                          preferred_element_type=jnp.float32)

        @pl.when(f == 0)
        def _():
            out_ref[...] = contrib + b2_ref[0]

        @pl.when(f != 0)
        def _():
            out_ref[...] += contrib


def _grouped_ffn(xs, w1, b1, w2, b2, tile_type, tile_valid):
    def f_act(m, f):
        return jnp.where(m % 2 == 0, f, _NF - 1 - f)

    grid_spec = pltpu.PrefetchScalarGridSpec(
        num_scalar_prefetch=2,
        grid=(_NUM_TILES, _NF),
        in_specs=[
            pl.BlockSpec((_TILE, _D // 2), lambda m, f, tt, vv: (m, 0)),
            pl.BlockSpec((1, _D, _FB), lambda m, f, tt, vv: (tt[m], 0, f_act(m, f))),
            pl.BlockSpec((1, 1, _FB), lambda m, f, tt, vv: (tt[m], 0, f_act(m, f))),
            pl.BlockSpec((1, _FB, _D), lambda m, f, tt, vv: (tt[m], f_act(m, f), 0)),
            pl.BlockSpec((1, 1, _D), lambda m, f, tt, vv: (tt[m], 0, 0)),
        ],
        out_specs=pl.BlockSpec((_TILE, _D), lambda m, f, tt, vv: (m, 0)),
    )
    return pl.pallas_call(
        _ffn_body,
        grid_spec=grid_spec,
        out_shape=jax.ShapeDtypeStruct((_P, _D), jnp.float32),
        compiler_params=pltpu.CompilerParams(
            dimension_semantics=("arbitrary", "arbitrary")),
    )(tile_type, tile_valid, xs, w1, b1, w2, b2)


def kernel(x, token_types, W1, b1, W2, b2):
    x2 = x.reshape(_T, _D)
    types = token_types.reshape(_T).astype(jnp.int32)

    oh = (types[:, None] == jnp.arange(_NT, dtype=jnp.int32)[None, :])
    csum = jnp.cumsum(oh.astype(jnp.int32), axis=0)
    counts = csum[-1]
    rank = jnp.take_along_axis(csum, types[:, None], axis=1)[:, 0] - 1

    starts = [jnp.int32(0)]
    for t in range(1, _NT):
        prev_end = starts[t - 1] + counts[t - 1]
        starts.append(((prev_end + _TILE - 1) // _TILE) * _TILE)
    starts = jnp.stack(starts)
    end = starts[-1] + counts[-1]

    slot = starts[types] + rank
    src_idx = jnp.zeros((_P,), jnp.int32).at[slot].set(
        jnp.arange(_T, dtype=jnp.int32))

    tile_start = jnp.arange(_NUM_TILES, dtype=jnp.int32) * _TILE
    tile_valid = (tile_start < end).astype(jnp.int32)
    tile_type = jnp.searchsorted(starts, tile_start, side="right").astype(
        jnp.int32) - 1
    tile_type = jnp.where(tile_valid == 1, tile_type, _NT - 1)

    xs = _sc_gather(_detile_cast(x2), src_idx, 48)
    ys = _grouped_ffn(xs, W1.astype(jnp.bfloat16), b1.reshape(_NT, 1, _F),
                      W2.astype(jnp.bfloat16), b2.reshape(_NT, 1, _D),
                      tile_type, tile_valid)
    out = _sc_gather(ys, slot, 32)
    return out.reshape(_B, _S, _D)


# speedup vs baseline: 1.2539x; 1.2539x over previous
import functools

import jax
import jax.numpy as jnp
from jax import lax
from jax.experimental import pallas as pl
from jax.experimental.pallas import tpu as pltpu
from jax.experimental.pallas import tpu_sc as plsc

_B = 2
_S = 4096
_T = _B * _S
_D = 1024
_F = 4096
_NT = 4

_TILE = 256
_P = _T + _NT * _TILE
_NUM_TILES = _P // _TILE
_FB = 2048
_NF = _F // _FB

_NC = 2
_NS = 16
_NW = _NC * _NS
_CHUNK = 32


def _sc_gather(table, idx, chunk):
    v, d = table.shape
    n = idx.shape[0]
    rows_per = n // _NW
    n_chunks = rows_per // chunk
    mesh = plsc.VectorSubcoreMesh(core_axis_name="c", subcore_axis_name="s")

    @functools.partial(
        pl.kernel,
        mesh=mesh,
        out_type=jax.ShapeDtypeStruct((n, d), table.dtype),
        scratch_types=[
            pltpu.VMEM((2, chunk), jnp.int32),
            pltpu.VMEM((2, chunk, d), table.dtype),
            pltpu.SemaphoreType.DMA,
            pltpu.SemaphoreType.DMA,
        ],
    )
    def gather_k(table_hbm, idx_hbm, out_hbm, idx_v, rows_v, sem0, sem1):
        wid = lax.axis_index("s") * _NC + lax.axis_index("c")
        base = wid * rows_per
        sems = (sem0, sem1)
        copies = [None, None]

        def start(c):
            b = c % 2
            pltpu.sync_copy(idx_hbm.at[pl.ds(base + c * chunk, chunk)],
                            idx_v.at[b])
            copies[b] = pltpu.async_copy(table_hbm.at[idx_v.at[b]],
                                         rows_v.at[b], sems[b])

        start(0)
        for c in range(n_chunks):
            b = c % 2
            if c + 1 < n_chunks:
                start(c + 1)
            copies[b].wait()
            pltpu.sync_copy(rows_v.at[b],
                            out_hbm.at[pl.ds(base + c * chunk, chunk)])

    return gather_k(table, idx)


def _sc_scatter(x2, slot):
    n, d = x2.shape
    rows_per = n // _NW
    chunk = 32
    n_chunks = rows_per // chunk
    mesh = plsc.VectorSubcoreMesh(core_axis_name="c", subcore_axis_name="s")

    @functools.partial(
        pl.kernel,
        mesh=mesh,
        out_type=jax.ShapeDtypeStruct((_P, d), x2.dtype),
        scratch_types=[
            pltpu.VMEM((2, chunk), jnp.int32),
            pltpu.VMEM((2, chunk, d), x2.dtype),
            pltpu.SemaphoreType.DMA,
            pltpu.SemaphoreType.DMA,
            pltpu.SemaphoreType.DMA,
        ],
    )
    def scatter_k(x_hbm, slot_hbm, out_hbm, idx_v, rows_v, sem0, sem1, sem_r):
        wid = lax.axis_index("s") * _NC + lax.axis_index("c")
        base = wid * rows_per
        sems = (sem0, sem1)
        reads = [None, None]
        scats = [None, None]

        def start_read(c):
            b = c % 2
            pltpu.sync_copy(slot_hbm.at[pl.ds(base + c * chunk, chunk)],
                            idx_v.at[b])
            reads[b] = pltpu.async_copy(
                x_hbm.at[pl.ds(base + c * chunk, chunk)], rows_v.at[b], sem_r)

        def wait_scat(b):
            if scats[b] is not None:
                scats[b].wait()
                scats[b] = None

        start_read(0)
        for c in range(n_chunks):
            b = c % 2
            reads[b].wait()
            if c + 1 < n_chunks:
                wait_scat(1 - b)
                start_read(c + 1)
            scats[b] = pltpu.async_copy(rows_v.at[b],
                                        out_hbm.at[idx_v.at[b]], sems[b])
        wait_scat(0)
        wait_scat(1)

    return scatter_k(x2, slot)


def _ffn_body(tt_ref, valid_ref, xs_ref, w1_ref, b1_ref, w2_ref, b2_ref,
              out_ref):
    m = pl.program_id(0)
    f = pl.program_id(1)

    @pl.when(valid_ref[m] == 1)
    def _():
        a = jnp.dot(xs_ref[...].astype(jnp.bfloat16), w1_ref[0],
                    preferred_element_type=jnp.float32) + b1_ref[0]
        h = 0.5 * a * (1.0 + lax.erf(a * 0.7071067811865476))
        contrib = jnp.dot(h.astype(jnp.bfloat16), w2_ref[0],
                          preferred_element_type=jnp.float32)

        @pl.when(f == 0)
        def _():
            out_ref[...] = contrib + b2_ref[0]

        @pl.when(f != 0)
        def _():
            out_ref[...] += contrib


def _grouped_ffn(xs, w1, b1, w2, b2, tile_type, tile_valid):
    def f_act(m, f):
        return jnp.where(m % 2 == 0, f, _NF - 1 - f)

    grid_spec = pltpu.PrefetchScalarGridSpec(
        num_scalar_prefetch=2,
        grid=(_NUM_TILES, _NF),
        in_specs=[
            pl.BlockSpec((_TILE, _D), lambda m, f, tt, vv: (m, 0)),
            pl.BlockSpec((1, _D, _FB), lambda m, f, tt, vv: (tt[m], 0, f_act(m, f))),
            pl.BlockSpec((1, 1, _FB), lambda m, f, tt, vv: (tt[m], 0, f_act(m, f))),
            pl.BlockSpec((1, _FB, _D), lambda m, f, tt, vv: (tt[m], f_act(m, f), 0)),
            pl.BlockSpec((1, 1, _D), lambda m, f, tt, vv: (tt[m], 0, 0)),
        ],
        out_specs=pl.BlockSpec((_TILE, _D), lambda m, f, tt, vv: (m, 0)),
    )
    return pl.pallas_call(
        _ffn_body,
        grid_spec=grid_spec,
        out_shape=jax.ShapeDtypeStruct((_P, _D), jnp.float32),
        compiler_params=pltpu.CompilerParams(
            dimension_semantics=("arbitrary", "arbitrary")),
    )(tile_type, tile_valid, xs, w1, b1, w2, b2)


def kernel(x, token_types, W1, b1, W2, b2):
    x2 = x.reshape(_T, _D)
    types = token_types.reshape(_T).astype(jnp.int32)

    oh = (types[:, None] == jnp.arange(_NT, dtype=jnp.int32)[None, :])
    csum = jnp.cumsum(oh.astype(jnp.int32), axis=0)
    counts = csum[-1]
    rank = jnp.take_along_axis(csum, types[:, None], axis=1)[:, 0] - 1

    starts = [jnp.int32(0)]
    for t in range(1, _NT):
        prev_end = starts[t - 1] + counts[t - 1]
        starts.append(((prev_end + _TILE - 1) // _TILE) * _TILE)
    starts = jnp.stack(starts)
    end = starts[-1] + counts[-1]

    slot = starts[types] + rank

    tile_start = jnp.arange(_NUM_TILES, dtype=jnp.int32) * _TILE
    tile_valid = (tile_start < end).astype(jnp.int32)
    tile_type = jnp.sum(
        (tile_start[:, None] >= starts[None, :]).astype(jnp.int32),
        axis=1) - 1
    tile_type = jnp.where(tile_valid == 1, tile_type, _NT - 1)

    xs = _sc_scatter(x2, slot)
    ys = _grouped_ffn(xs, W1.astype(jnp.bfloat16), b1.reshape(_NT, 1, _F),
                      W2.astype(jnp.bfloat16), b2.reshape(_NT, 1, _D),
                      tile_type, tile_valid)
    out = _sc_gather(ys, slot, 32)
    return out.reshape(_B, _S, _D)
